# Initial kernel scaffold; baseline (speedup 1.0000x reference)
import jax
import jax.numpy as jnp
from jax.experimental import pallas as pl


def kernel(anchors, deltas, scores):
    raise NotImplementedError("write your pallas kernel here")


# speedup vs baseline: 36.4939x; 36.4939x over previous
import math

import jax
import jax.numpy as jnp
from jax.experimental import pallas as pl

_N = 20000
_NPAD = 20480
_NROWS = 160
_K1 = 2000
_KPAD = 2048
_NBLK = 16
_B = 128
_K2 = 1000
_T = 0.7
_CLAMP = math.log(1000.0 / 16.0)
_IMG = 1024.0


def _dot(a, b):
    return jax.lax.dot_general(a, b, (((1,), (0,)), ((), ())),
                               preferred_element_type=jnp.float32)


def _decode_body(ax1, ay1, ax2, ay2, dx, dy, dw, dh, sc,
                 ox1, oy1, ox2, oy2, os):
    w = ax2[...] - ax1[...]
    h = ay2[...] - ay1[...]
    cx = ax1[...] + 0.5 * w
    cy = ay1[...] + 0.5 * h
    ddw = jnp.minimum(dw[...], _CLAMP)
    ddh = jnp.minimum(dh[...], _CLAMP)
    px = dx[...] * w + cx
    py = dy[...] * h + cy
    pw = jnp.exp(ddw) * w
    ph = jnp.exp(ddh) * h
    x1 = jnp.clip(px - 0.5 * pw, 0.0, _IMG)
    y1 = jnp.clip(py - 0.5 * ph, 0.0, _IMG)
    x2 = jnp.clip(px + 0.5 * pw, 0.0, _IMG)
    y2 = jnp.clip(py + 0.5 * ph, 0.0, _IMG)
    valid = ((x2 - x1) > 0.0) & ((y2 - y1) > 0.0)
    ox1[...] = x1
    oy1[...] = y1
    ox2[...] = x2
    oy2[...] = y2
    os[...] = jnp.where(valid, sc[...], -jnp.inf)


def _nms_body(x1c, y1c, x2c, y2c, x1r, y1r, x2r, y2r, keep_out):
    cx1, cy1, cx2, cy2 = x1c[...], y1c[...], x2c[...], y2c[...]
    area_c = (cx2 - cx1) * (cy2 - cy1)
    colg = jax.lax.broadcasted_iota(jnp.int32, (1, _KPAD), 1)
    keep = colg < _K1

    li = jax.lax.broadcasted_iota(jnp.int32, (_B, _B), 0)
    lj = jax.lax.broadcasted_iota(jnp.int32, (_B, _B), 1)
    upper = li < lj

    for r in range(_NBLK):
        rx1 = x1r[:, r:r + 1]
        ry1 = y1r[:, r:r + 1]
        rx2 = x2r[:, r:r + 1]
        ry2 = y2r[:, r:r + 1]
        area_r = (rx2 - rx1) * (ry2 - ry1)
        wx = jnp.maximum(jnp.minimum(rx2, cx2) - jnp.maximum(rx1, cx1), 0.0)
        wy = jnp.maximum(jnp.minimum(ry2, cy2) - jnp.maximum(ry1, cy1), 0.0)
        inter = wx * wy
        union = area_r + area_c - inter
        iou = inter / jnp.maximum(union, 1e-9)
        mf = (iou > _T).astype(jnp.float32)
        mbb = jnp.where(upper, mf[:, r * _B:(r + 1) * _B], 0.0)

        active0 = keep[:, r * _B:(r + 1) * _B].astype(jnp.float32)

        def cond(c):
            return jnp.logical_not(c[2])

        def body(c):
            act, _, _ = c
            pse = _dot(act, mbb) > 0.0
            knew = jnp.where(pse, 0.0, act)
            ns = _dot(knew, mbb) > 0.0
            act2 = jnp.where(ns, 0.0, act)
            done = jnp.all(act2 == act)
            return act2, knew, done

        _, kept, _ = jax.lax.while_loop(
            cond, body, (active0, active0, jnp.bool_(False)))

        supp_later = (_dot(kept, mf) > 0.0) & (colg >= (r + 1) * _B)
        su_blk = (active0 > 0.0) & jnp.logical_not(kept > 0.0)
        parts = []
        if r > 0:
            parts.append(jnp.zeros((1, r * _B), jnp.bool_))
        parts.append(su_blk)
        if r < _NBLK - 1:
            parts.append(jnp.zeros((1, (_NBLK - 1 - r) * _B), jnp.bool_))
        su_full = jnp.concatenate(parts, axis=1)
        keep = keep & jnp.logical_not(supp_later | su_full)

    keep_out[...] = keep.astype(jnp.float32)


def _decode(planes):
    outs = [jax.ShapeDtypeStruct((_NROWS, 128), jnp.float32)] * 5
    return pl.pallas_call(_decode_body, out_shape=outs)(*planes)


def _nms(cols, rows):
    return pl.pallas_call(
        _nms_body,
        out_shape=jax.ShapeDtypeStruct((1, _KPAD), jnp.float32),
    )(*cols, *rows)


def kernel(anchors, deltas, scores):
    a = jnp.pad(anchors, ((0, _NPAD - _N), (0, 0)))
    d = jnp.pad(deltas, ((0, _NPAD - _N), (0, 0)))
    s = jnp.pad(scores, (0, _NPAD - _N))
    planes = [a[:, i].reshape(_NROWS, 128) for i in range(4)]
    planes += [d[:, i].reshape(_NROWS, 128) for i in range(4)]
    planes.append(s.reshape(_NROWS, 128))
    x1, y1, x2, y2, sm = _decode(planes)

    vals, idx = jax.lax.top_k(sm.reshape(-1), _K1)
    g = [f.reshape(-1)[idx] for f in (x1, y1, x2, y2)]

    pad = _KPAD - _K1
    cols = [jnp.pad(f, (0, pad)).reshape(1, _KPAD) for f in g]
    rows = [jnp.pad(f, (0, pad)).reshape(_NBLK, _B).T for f in g]
    keep = _nms(cols, rows)

    keepb = keep[0, :_K1] > 0.0
    masked = jnp.where(keepb, vals, -jnp.inf)
    _, idx2 = jax.lax.top_k(masked, _K2)
    out = jnp.stack([g[0][idx2], g[1][idx2], g[2][idx2], g[3][idx2],
                     vals[idx2]], axis=1)
    return out


# speedup vs baseline: 54.4987x; 1.4934x over previous
import math

import jax
import jax.numpy as jnp
from jax.experimental import pallas as pl

_N = 20000
_NPAD = 20480
_NROWS = 160
_K1 = 2000
_KPAD = 2048
_NBLK = 16
_B = 128
_K2 = 1000
_T = 0.7
_CLAMP = math.log(1000.0 / 16.0)
_IMG = 1024.0


def _dot(a, b):
    return jax.lax.dot_general(a, b, (((1,), (0,)), ((), ())),
                               preferred_element_type=jnp.float32)


def _decode_body(ax1, ay1, ax2, ay2, dx, dy, dw, dh, sc,
                 ox1, oy1, ox2, oy2, os):
    w = ax2[...] - ax1[...]
    h = ay2[...] - ay1[...]
    cx = ax1[...] + 0.5 * w
    cy = ay1[...] + 0.5 * h
    ddw = jnp.minimum(dw[...], _CLAMP)
    ddh = jnp.minimum(dh[...], _CLAMP)
    px = dx[...] * w + cx
    py = dy[...] * h + cy
    pw = jnp.exp(ddw) * w
    ph = jnp.exp(ddh) * h
    x1 = jnp.clip(px - 0.5 * pw, 0.0, _IMG)
    y1 = jnp.clip(py - 0.5 * ph, 0.0, _IMG)
    x2 = jnp.clip(px + 0.5 * pw, 0.0, _IMG)
    y2 = jnp.clip(py + 0.5 * ph, 0.0, _IMG)
    valid = ((x2 - x1) > 0.0) & ((y2 - y1) > 0.0)
    ox1[...] = x1
    oy1[...] = y1
    ox2[...] = x2
    oy2[...] = y2
    os[...] = jnp.where(valid, sc[...], -jnp.inf)


def _nms_body(x1c, y1c, x2c, y2c, x1r, y1r, x2r, y2r, vr, out40):
    cx1, cy1, cx2, cy2 = x1c[...], y1c[...], x2c[...], y2c[...]
    area_c = (cx2 - cx1) * (cy2 - cy1)
    colg = jax.lax.broadcasted_iota(jnp.int32, (1, _KPAD), 1)
    keep = colg < _K1

    li = jax.lax.broadcasted_iota(jnp.int32, (_B, _B), 0)
    lj = jax.lax.broadcasted_iota(jnp.int32, (_B, _B), 1)
    upper = li < lj

    for r in range(_NBLK):
        rx1 = x1r[:, r:r + 1]
        ry1 = y1r[:, r:r + 1]
        rx2 = x2r[:, r:r + 1]
        ry2 = y2r[:, r:r + 1]
        area_r = (rx2 - rx1) * (ry2 - ry1)
        wx = jnp.maximum(jnp.minimum(rx2, cx2) - jnp.maximum(rx1, cx1), 0.0)
        wy = jnp.maximum(jnp.minimum(ry2, cy2) - jnp.maximum(ry1, cy1), 0.0)
        inter = wx * wy
        union = area_r + area_c - inter
        iou = inter / jnp.maximum(union, 1e-9)
        mf = (iou > _T).astype(jnp.float32)
        mbb = jnp.where(upper, mf[:, r * _B:(r + 1) * _B], 0.0)

        active0 = keep[:, r * _B:(r + 1) * _B].astype(jnp.float32)

        def cond(c):
            return jnp.logical_not(c[2])

        def body(c):
            act, _, _ = c
            pse = _dot(act, mbb) > 0.0
            knew = jnp.where(pse, 0.0, act)
            ns = _dot(knew, mbb) > 0.0
            act2 = jnp.where(ns, 0.0, act)
            done = jnp.all(act2 == act)
            return act2, knew, done

        _, kept, _ = jax.lax.while_loop(
            cond, body, (active0, active0, jnp.bool_(False)))

        supp_later = (_dot(kept, mf) > 0.0) & (colg >= (r + 1) * _B)
        su_blk = (active0 > 0.0) & jnp.logical_not(kept > 0.0)
        parts = []
        if r > 0:
            parts.append(jnp.zeros((1, r * _B), jnp.bool_))
        parts.append(su_blk)
        if r < _NBLK - 1:
            parts.append(jnp.zeros((1, (_NBLK - 1 - r) * _B), jnp.bool_))
        su_full = jnp.concatenate(parts, axis=1)
        keep = keep & jnp.logical_not(supp_later | su_full)

    kept16 = jnp.reshape(keep.astype(jnp.float32), (_NBLK, _B))
    lidx = jax.lax.broadcasted_iota(jnp.int32, (_B, _B), 0)
    lcol = jax.lax.broadcasted_iota(jnp.int32, (_B, _B), 1)
    tu = (lidx <= lcol).astype(jnp.float32)
    bi = jax.lax.broadcasted_iota(jnp.int32, (_NBLK, _NBLK), 0)
    bj = jax.lax.broadcasted_iota(jnp.int32, (_NBLK, _NBLK), 1)
    s16 = (bj < bi).astype(jnp.float32)

    def _explusive_prefix(m16):
        inc = _dot(m16, tu)
        rt = inc[:, _B - 1:_B]
        off = _dot(s16, rt)
        return inc - m16 + off

    exk = _explusive_prefix(kept16)
    nk16 = 1.0 - kept16
    exn = _explusive_prefix(nk16)
    ktot = _dot(jnp.ones((1, _NBLK), jnp.float32),
                _dot(kept16, tu)[:, _B - 1:_B])
    pos2f = jnp.where(kept16 > 0.0, exk, ktot + exn)
    pos2 = pos2f.astype(jnp.int32)
    lo16 = pos2 & (_B - 1)
    hi16 = pos2 >> 7
    valid16 = pos2 < 1024
    hi_rows = jnp.transpose(pos2f).astype(jnp.int32) >> 7

    iota_cl = jax.lax.broadcasted_iota(jnp.int32, (_B, _B), 0)
    iota_l = jax.lax.broadcasted_iota(jnp.int32, (_B, _B), 1)
    iota_h8 = jax.lax.broadcasted_iota(jnp.int32, (_B, 8), 1)
    acc = jnp.zeros((_B, 40), jnp.float32)
    for r in range(_NBLK):
        lo_row = lo16[r:r + 1, :]
        v_row = valid16[r:r + 1, :]
        ohlo_t = ((iota_cl == lo_row) & v_row).astype(jnp.float32)
        hi_col = hi_rows[:, r:r + 1]
        ohhi = (hi_col == iota_h8).astype(jnp.float32)
        dall = jnp.concatenate(
            [f[:, r:r + 1] * ohhi for f in (x1r, y1r, x2r, y2r, vr)], axis=1)
        acc = acc + _dot(ohlo_t, dall)
    out40[...] = acc


def _decode(planes):
    outs = [jax.ShapeDtypeStruct((_NROWS, 128), jnp.float32)] * 5
    return pl.pallas_call(_decode_body, out_shape=outs)(*planes)


def _nms(cols, rows, vr):
    return pl.pallas_call(
        _nms_body,
        out_shape=jax.ShapeDtypeStruct((_B, 40), jnp.float32),
    )(*cols, *rows, vr)


def kernel(anchors, deltas, scores):
    a = jnp.pad(anchors, ((0, _NPAD - _N), (0, 0)))
    d = jnp.pad(deltas, ((0, _NPAD - _N), (0, 0)))
    s = jnp.pad(scores, (0, _NPAD - _N))
    planes = [a[:, i].reshape(_NROWS, 128) for i in range(4)]
    planes += [d[:, i].reshape(_NROWS, 128) for i in range(4)]
    planes.append(s.reshape(_NROWS, 128))
    x1, y1, x2, y2, sm = _decode(planes)

    vals, idx = jax.lax.top_k(sm.reshape(-1), _K1)
    g = [f.reshape(-1)[idx] for f in (x1, y1, x2, y2)]

    pad = _KPAD - _K1
    cols = [jnp.pad(f, (0, pad)).reshape(1, _KPAD) for f in g]
    rows = [jnp.pad(f, (0, pad)).reshape(_NBLK, _B).T for f in g]
    vr = jnp.maximum(jnp.pad(vals, (0, pad), constant_values=-jnp.inf),
                     jnp.float32(-3.4e38)).reshape(_NBLK, _B).T
    out40 = _nms(cols, rows, vr)

    fields = [out40[:, 8 * f:8 * (f + 1)].T.reshape(-1)[:_K2]
              for f in range(5)]
    return jnp.stack(fields, axis=1)
